# Initial kernel scaffold; baseline (speedup 1.0000x reference)
import jax
import jax.numpy as jnp
from jax.experimental import pallas as pl


def kernel(x, wg_outer, wg_inner, w1, b1, w2, b2):
    raise NotImplementedError("write your pallas kernel here")


# speedup vs baseline: 2.1544x; 2.1544x over previous
import functools

import jax
import jax.numpy as jnp
from jax.experimental import pallas as pl
from jax.experimental.pallas import tpu as pltpu

N = 2048
D = 768
H = 768
G = 2
M = 4
NE = G * M
BLK = 256


def _gates_for_group(il, pout):
    idx = jax.lax.broadcasted_iota(jnp.int32, il.shape, 1)
    v1 = jnp.max(il, axis=1, keepdims=True)
    i1 = jnp.min(jnp.where(il == v1, idx, M), axis=1, keepdims=True)
    il2 = jnp.where(idx == i1, -jnp.inf, il)
    v2 = jnp.max(il2, axis=1, keepdims=True)
    i2 = jnp.min(jnp.where(il2 == v2, idx, M), axis=1, keepdims=True)
    e2 = jnp.exp(v2 - v1)
    denom = 1.0 + e2
    p1 = 1.0 / denom
    p2 = e2 / denom
    gates = jnp.where(idx == i1, p1, 0.0) + jnp.where(idx == i2, p2, 0.0)
    return gates * pout


def _moe_body(x_ref, wg_ref, w1_ref, b1_ref, w2_ref, b2_ref, out_ref):
    x = x_ref[...]
    lg = jnp.dot(x, wg_ref[...], preferred_element_type=jnp.float32)

    o = lg[:, 0:G]
    om = jnp.max(o, axis=1, keepdims=True)
    oe = jnp.exp(o - om)
    pout = oe / jnp.sum(oe, axis=1, keepdims=True)

    gcols = [
        _gates_for_group(lg[:, G + M * g: G + M * (g + 1)], pout[:, g:g + 1])
        for g in range(G)
    ]

    acc = jnp.zeros((BLK, D), jnp.float32)
    for g in range(G):
        for m in range(M):
            e = M * g + m
            h = jnp.dot(x, w1_ref[e], preferred_element_type=jnp.float32)
            h = jnp.maximum(h + b1_ref[e], 0.0).astype(jnp.bfloat16)
            y = jnp.dot(h, w2_ref[e], preferred_element_type=jnp.float32)
            y = y + b2_ref[e]
            acc = acc + gcols[g][:, m:m + 1] * y
    out_ref[...] = acc


@jax.jit
def kernel(x, wg_outer, wg_inner, w1, b1, w2, b2):
    xb = x.astype(jnp.bfloat16)
    wg_cat = jnp.concatenate(
        [wg_outer] + [wg_inner[g] for g in range(G)], axis=1)
    wg_cat = jnp.pad(wg_cat, ((0, 0), (0, 16 - (G + G * M))))
    wg_cat = wg_cat.astype(jnp.bfloat16)
    w1r = w1.reshape(NE, D, H).astype(jnp.bfloat16)
    w2r = w2.reshape(NE, H, D).astype(jnp.bfloat16)
    b1r = b1.reshape(NE, H)
    b2r = b2.reshape(NE, D)

    grid = (N // BLK,)
    out = pl.pallas_call(
        _moe_body,
        grid=grid,
        in_specs=[
            pl.BlockSpec((BLK, D), lambda b: (b, 0)),
            pl.BlockSpec((D, 16), lambda b: (0, 0)),
            pl.BlockSpec((NE, D, H), lambda b: (0, 0, 0)),
            pl.BlockSpec((NE, H), lambda b: (0, 0)),
            pl.BlockSpec((NE, H, D), lambda b: (0, 0, 0)),
            pl.BlockSpec((NE, D), lambda b: (0, 0)),
        ],
        out_specs=pl.BlockSpec((BLK, D), lambda b: (b, 0)),
        out_shape=jax.ShapeDtypeStruct((N, D), jnp.float32),
        compiler_params=pltpu.CompilerParams(
            dimension_semantics=("parallel",),
        ),
    )(xb, wg_cat, w1r, b1r, w2r, b2r)
    return out
